# baseline (speedup 1.0000x reference)
import functools
import math

import jax
import jax.numpy as jnp
from jax import lax
from jax.experimental import pallas as pl
from jax.experimental.pallas import tpu as pltpu
from jax.experimental.pallas import tpu_sc as plsc

_VOCAB = 1000000
_DIM = 64
_B = 16384
_L = 200
_N = _B * _L
_NC = 2
_NS = 16
_NW = _NC * _NS
_PER_W = _N // _NW
_CHUNK = 800
_STEPS = _PER_W // _CHUNK
_SCALE = math.sqrt(_DIM)

_mesh = plsc.VectorSubcoreMesh(core_axis_name="c", subcore_axis_name="s")


@functools.partial(
    pl.kernel,
    out_type=jax.ShapeDtypeStruct((_N, _DIM), jnp.float32),
    mesh=_mesh,
    scratch_types=[
        pltpu.VMEM((_CHUNK,), jnp.int32),
        pltpu.VMEM((_CHUNK, _DIM), jnp.float32),
        pltpu.SemaphoreType.DMA,
    ],
    compiler_params=pltpu.CompilerParams(use_tc_tiling_on_sc=False),
)
def _embed_gather(table_hbm, idx_hbm, out_hbm, idx_v, rows_v, sem):
    wid = lax.axis_index("s") * _NC + lax.axis_index("c")
    base = wid * _PER_W

    def step(i, carry):
        off = base + i * _CHUNK
        pltpu.sync_copy(idx_hbm.at[pl.ds(off, _CHUNK)], idx_v)
        pltpu.async_copy(table_hbm.at[idx_v], rows_v, sem).wait()

        def scale_row(r, c2):
            for c in range(_DIM // 16):
                sl = pl.ds(c * 16, 16)
                rows_v[r, sl] = rows_v[r, sl] * _SCALE
            return c2

        lax.fori_loop(0, _CHUNK, scale_row, 0, unroll=2)
        pltpu.sync_copy(rows_v, out_hbm.at[pl.ds(off, _CHUNK)])
        return carry

    lax.fori_loop(0, _STEPS, step, 0)


def kernel(tokens, table):
    flat = tokens.reshape(_N)
    out = _embed_gather(table, flat)
    return out.reshape(_B, _L, _DIM)


# speedup vs baseline: 1.1067x; 1.1067x over previous
import functools
import math

import jax
import jax.numpy as jnp
from jax import lax
from jax.experimental import pallas as pl
from jax.experimental.pallas import tpu as pltpu
from jax.experimental.pallas import tpu_sc as plsc

_VOCAB = 1000000
_DIM = 64
_B = 16384
_L = 200
_N = _B * _L
_NC = 2
_NS = 16
_NW = _NC * _NS
_PER_W = _N // _NW
_CHUNK = 800
_STEPS = _PER_W // _CHUNK
_SCALE = math.sqrt(_DIM)

_mesh = plsc.VectorSubcoreMesh(core_axis_name="c", subcore_axis_name="s")


@functools.partial(
    pl.kernel,
    out_type=jax.ShapeDtypeStruct((_N, _DIM), jnp.float32),
    mesh=_mesh,
    scratch_types=[
        pltpu.VMEM((_CHUNK,), jnp.int32),
        pltpu.VMEM((_CHUNK,), jnp.int32),
        pltpu.VMEM((_CHUNK, _DIM), jnp.float32),
        pltpu.VMEM((_CHUNK, _DIM), jnp.float32),
        pltpu.SemaphoreType.DMA,
        pltpu.SemaphoreType.DMA,
        pltpu.SemaphoreType.DMA,
        pltpu.SemaphoreType.DMA,
    ],
    compiler_params=pltpu.CompilerParams(use_tc_tiling_on_sc=False),
)
def _embed_gather(table_hbm, idx_hbm, out_hbm,
                  idx0, idx1, rows0, rows1, g0, g1, s0, s1):
    wid = lax.axis_index("s") * _NC + lax.axis_index("c")
    base = wid * _PER_W
    idx_v = (idx0, idx1)
    rows_v = (rows0, rows1)
    gsem = (g0, g1)
    ssem = (s0, s1)

    def chunk_off(i):
        return base + i * _CHUNK

    pltpu.sync_copy(idx_hbm.at[pl.ds(chunk_off(0), _CHUNK)], idx0)
    pltpu.async_copy(table_hbm.at[idx0], rows0, g0)

    def scale_rows(rv):
        def scale_row(r, carry):
            for c in range(_DIM // 16):
                sl = pl.ds(c * 16, 16)
                rv[r, sl] = rv[r, sl] * _SCALE
            return carry
        lax.fori_loop(0, _CHUNK, scale_row, 0, unroll=4)

    def outer(g, carry):
        for b in range(2):
            i = 2 * g + b
            nb = 1 - b
            pltpu.make_async_copy(table_hbm.at[idx_v[b]], rows_v[b], gsem[b]).wait()
            if b == 0:
                @pl.when(g > 0)
                def _wait_prev_store():
                    pltpu.make_async_copy(
                        rows_v[nb], out_hbm.at[pl.ds(chunk_off(0), _CHUNK)], ssem[nb]
                    ).wait()
                pltpu.sync_copy(idx_hbm.at[pl.ds(chunk_off(i + 1), _CHUNK)], idx_v[nb])
                pltpu.async_copy(table_hbm.at[idx_v[nb]], rows_v[nb], gsem[nb])
            else:
                @pl.when(g < _STEPS // 2 - 1)
                def _prefetch():
                    pltpu.make_async_copy(
                        rows_v[nb], out_hbm.at[pl.ds(chunk_off(0), _CHUNK)], ssem[nb]
                    ).wait()
                    pltpu.sync_copy(
                        idx_hbm.at[pl.ds(chunk_off(i + 1), _CHUNK)], idx_v[nb])
                    pltpu.async_copy(table_hbm.at[idx_v[nb]], rows_v[nb], gsem[nb])
            scale_rows(rows_v[b])
            pltpu.async_copy(
                rows_v[b], out_hbm.at[pl.ds(chunk_off(i), _CHUNK)], ssem[b])
        return carry

    lax.fori_loop(0, _STEPS // 2, outer, 0)
    pltpu.make_async_copy(rows0, out_hbm.at[pl.ds(base, _CHUNK)], s0).wait()
    pltpu.make_async_copy(rows1, out_hbm.at[pl.ds(base, _CHUNK)], s1).wait()


def kernel(tokens, table):
    flat = tokens.reshape(_N)
    out = _embed_gather(table, flat)
    return out.reshape(_B, _L, _DIM)
